# baseline (speedup 1.0000x reference)
import functools

import jax
import jax.numpy as jnp
from jax import lax
from jax.experimental import pallas as pl
from jax.experimental.pallas import tpu as pltpu
from jax.experimental.pallas import tpu_sc as plsc

VOCAB = 1000000
DIM = 16
EPS = 1e-05
B = 16384
L = 50
N = B * L

NC, NS, LANES = 2, 16, 16
NW = NC * NS

TBLK = 512
VMAIN = (VOCAB // TBLK) * TBLK
NTAIL = VOCAB - VMAIN
MAXOFF_A = VMAIN - TBLK
BPW_A = 62


def _tr_body(tt_hbm, tail_hbm, tl_hbm, blkA, blkB, trA, trB, tailv,
             s_iA, s_iB, s_oA, s_oB):
    cid = lax.axis_index("c")
    sid = lax.axis_index("s")
    wid = sid * NC + cid
    b0 = wid * BPW_A

    def off_of(b):
        return pl.multiple_of(jnp.minimum(b * TBLK, MAXOFF_A), TBLK)

    def issue_in(b, blk, sem):
        return pltpu.async_copy(tt_hbm.at[:, pl.ds(off_of(b), TBLK)], blk, sem)

    def wait_in(blk, sem):
        pltpu.make_async_copy(tt_hbm.at[:, pl.ds(0, TBLK)], blk, sem).wait()

    scat = lax.iota(jnp.int32, LANES) * DIM

    def transpose(blk, tr):
        for d in range(DIM):
            segs = [blk[d, pl.ds(vg * LANES, LANES)] for vg in range(TBLK // LANES)]
            for vg in range(TBLK // LANES):
                plsc.store_scatter(tr, [scat + (vg * LANES * DIM + d)], segs[vg])

    def issue_out(b, tr, sem):
        return pltpu.async_copy(tr, tl_hbm.at[pl.ds(off_of(b) * DIM, TBLK * DIM)], sem)

    issue_in(b0, blkA, s_iA)

    def pair(i, c):
        be = b0 + 2 * i
        issue_in(be + 1, blkB, s_iB)
        wait_in(blkA, s_iA)
        transpose(blkA, trA)
        oA = issue_out(be, trA, s_oA)
        issue_in(be + 2, blkA, s_iA)
        wait_in(blkB, s_iB)
        transpose(blkB, trB)
        oB = issue_out(be + 1, trB, s_oB)
        oA.wait()
        oB.wait()
        return c

    @pl.when(wid == 0)
    def _():
        pltpu.sync_copy(tail_hbm, tailv)
        pltpu.sync_copy(tailv, tl_hbm.at[pl.ds(VMAIN * DIM, NTAIL * DIM)])

    lax.fori_loop(0, BPW_A // 2, pair, 0)
    wait_in(blkA, s_iA)


_tr_kernel = pl.kernel(
    _tr_body,
    out_type=jax.ShapeDtypeStruct((VOCAB * DIM,), jnp.float32),
    mesh=plsc.VectorSubcoreMesh(core_axis_name="c", subcore_axis_name="s",
                                num_cores=NC, num_subcores=NS),
    scratch_types=[
        pltpu.VMEM((DIM, TBLK), jnp.float32),
        pltpu.VMEM((DIM, TBLK), jnp.float32),
        pltpu.VMEM((TBLK * DIM,), jnp.float32),
        pltpu.VMEM((TBLK * DIM,), jnp.float32),
        pltpu.VMEM((NTAIL * DIM,), jnp.float32),
        pltpu.SemaphoreType.DMA,
        pltpu.SemaphoreType.DMA,
        pltpu.SemaphoreType.DMA,
        pltpu.SemaphoreType.DMA,
    ],
    compiler_params=pltpu.CompilerParams(needs_layout_passes=False,
                                         use_tc_tiling_on_sc=True),
)

CH = 1280
PER_W = N // NW
NCH = PER_W // CH


def _sc_body(lidx_hbm, ridx_hbm, table_hbm,
             uu_hbm, uv_hbm, vv_hbm, al_hbm, be_hbm, ga_hbm,
             liA, riA, liB, riB, uA, vA, uB, vB,
             ouuA, ouvA, ovvA, oalA, obeA, ogaA,
             ouuB, ouvB, ovvB, oalB, obeB, ogaB,
             s_uA, s_vA, s_uB, s_vB, s_oA, s_oB):
    cid = lax.axis_index("c")
    sid = lax.axis_index("s")
    wid = sid * NC + cid
    base_w = wid * PER_W

    def base_of(c):
        return base_w + jnp.minimum(c, NCH - 1) * CH

    H = CH // 2

    def fetch(c, li, ri, u_v, v_v, su, sv):
        base = base_of(c)
        pltpu.sync_copy(lidx_hbm.at[pl.ds(base, CH)], li)
        pltpu.sync_copy(ridx_hbm.at[pl.ds(base, CH)], ri)
        pltpu.async_copy(table_hbm.at[li.at[pl.ds(0, H)]], u_v.at[pl.ds(0, H), :], su)
        pltpu.async_copy(table_hbm.at[li.at[pl.ds(H, H)]], u_v.at[pl.ds(H, H), :], su)
        pltpu.async_copy(table_hbm.at[ri.at[pl.ds(0, H)]], v_v.at[pl.ds(0, H), :], sv)
        pltpu.async_copy(table_hbm.at[ri.at[pl.ds(H, H)]], v_v.at[pl.ds(H, H), :], sv)

    def wait_fetch(u_v, v_v, su, sv):
        pltpu.make_async_copy(table_hbm.at[pl.ds(0, H), :], u_v.at[pl.ds(0, H), :], su).wait()
        pltpu.make_async_copy(table_hbm.at[pl.ds(0, H), :], u_v.at[pl.ds(H, H), :], su).wait()
        pltpu.make_async_copy(table_hbm.at[pl.ds(0, H), :], v_v.at[pl.ds(0, H), :], sv).wait()
        pltpu.make_async_copy(table_hbm.at[pl.ds(0, H), :], v_v.at[pl.ds(H, H), :], sv).wait()

    def compute(u_v, v_v, ouu, ouv, ovv, oal, obe, oga):
        def grp(g, c2):
            rows = pl.multiple_of(g * LANES, LANES) + lax.iota(jnp.int32, LANES)
            uu = jnp.zeros((LANES,), jnp.float32)
            vv = jnp.zeros((LANES,), jnp.float32)
            uv = jnp.zeros((LANES,), jnp.float32)
            for d in range(DIM):
                dcol = jnp.full((LANES,), d, jnp.int32)
                cu = plsc.load_gather(u_v, [rows, dcol])
                cv = plsc.load_gather(v_v, [rows, dcol])
                uu = uu + cu * cu
                vv = vv + cv * cv
                uv = uv + cu * cv
            alpha = 1.0 - uu
            alpha = jnp.where(alpha <= 0.0, EPS, alpha)
            beta = 1.0 - vv
            beta = jnp.where(beta <= 0.0, EPS, beta)
            gamma = 1.0 + 2.0 * (uu - 2.0 * uv + vv) / alpha / beta
            gamma = jnp.maximum(gamma, 1.0)
            s = pl.ds(pl.multiple_of(g * LANES, LANES), LANES)
            ouu[s] = uu
            ouv[s] = uv
            ovv[s] = vv
            oal[s] = alpha
            obe[s] = beta
            oga[s] = gamma
            return c2

        lax.fori_loop(0, CH // LANES, grp, 0)

    def issue_out(c, bufs, sem):
        dst = pl.ds(base_of(c), CH)
        cps = []
        for buf, hbm in zip(bufs, (uu_hbm, uv_hbm, vv_hbm, al_hbm, be_hbm, ga_hbm)):
            cps.append(pltpu.async_copy(buf, hbm.at[dst], sem))
        return cps

    bufsA = (ouuA, ouvA, ovvA, oalA, obeA, ogaA)
    bufsB = (ouuB, ouvB, ovvB, oalB, obeB, ogaB)

    fetch(0, liA, riA, uA, vA, s_uA, s_vA)

    def body(i, c):
        c0 = 2 * i
        fetch(c0 + 1, liB, riB, uB, vB, s_uB, s_vB)
        wait_fetch(uA, vA, s_uA, s_vA)
        compute(uA, vA, *bufsA)
        cpsA = issue_out(c0, bufsA, s_oA)
        fetch(c0 + 2, liA, riA, uA, vA, s_uA, s_vA)
        wait_fetch(uB, vB, s_uB, s_vB)
        compute(uB, vB, *bufsB)
        cpsB = issue_out(c0 + 1, bufsB, s_oB)
        for cp in cpsA:
            cp.wait()
        for cp in cpsB:
            cp.wait()
        return c

    lax.fori_loop(0, NCH // 2, body, 0)
    wait_fetch(uA, vA, s_uA, s_vA)


_sc_kernel = pl.kernel(
    _sc_body,
    out_type=tuple(jax.ShapeDtypeStruct((N,), jnp.float32) for _ in range(6)),
    mesh=plsc.VectorSubcoreMesh(core_axis_name="c", subcore_axis_name="s",
                                num_cores=NC, num_subcores=NS),
    scratch_types=(
        [pltpu.VMEM((CH,), jnp.int32)] * 4
        + [pltpu.VMEM((CH, DIM), jnp.float32)] * 4
        + [pltpu.VMEM((CH,), jnp.float32)] * 12
        + [pltpu.SemaphoreType.DMA] * 6
    ),
    compiler_params=pltpu.CompilerParams(needs_layout_passes=False,
                                         use_tc_tiling_on_sc=False),
)


def _arcosh_body(g_ref, d_ref):
    g = g_ref[...]
    d_ref[...] = jnp.log(g + jnp.sqrt(g * g - 1.0))


def kernel(left_idx, right_idx, table):
    lidx = left_idx.reshape(N).astype(jnp.int32)
    ridx = right_idx.reshape(N).astype(jnp.int32)
    tail = lax.slice(table, (VMAIN, 0), (VOCAB, DIM)).reshape(NTAIL * DIM)
    tl = _tr_kernel(table.T, tail)
    uu, uv, vv, alpha, beta, gamma = _sc_kernel(lidx, ridx, tl.reshape(VOCAB, DIM))
    g2d = gamma.reshape(N // 128, 128)
    dists = pl.pallas_call(
        _arcosh_body,
        out_shape=jax.ShapeDtypeStruct((N // 128, 128), jnp.float32),
    )(g2d)
    shp = (B, L)
    return (uu.reshape(shp), uv.reshape(shp), vv.reshape(shp),
            alpha.reshape(shp), beta.reshape(shp), gamma.reshape(shp),
            dists.reshape(shp))


# speedup vs baseline: 1.0348x; 1.0348x over previous
import functools

import jax
import jax.numpy as jnp
from jax import lax
from jax.experimental import pallas as pl
from jax.experimental.pallas import tpu as pltpu
from jax.experimental.pallas import tpu_sc as plsc

VOCAB = 1000000
DIM = 16
EPS = 1e-05
B = 16384
L = 50
N = B * L

NC, NS, LANES = 2, 16, 16
NW = NC * NS

TBLK = 256
VMAIN = (VOCAB // TBLK) * TBLK
NTAIL = VOCAB - VMAIN
MAXOFF_A = VMAIN - TBLK
BPW_A = 124


def _tr_body(tt_hbm, tail_hbm, tl_hbm, blkA, blkB, trA, trB, tailv,
             s_iA, s_iB, s_oA, s_oB):
    cid = lax.axis_index("c")
    sid = lax.axis_index("s")
    wid = sid * NC + cid
    b0 = wid * BPW_A

    def off_of(b):
        return pl.multiple_of(jnp.minimum(b * TBLK, MAXOFF_A), TBLK)

    def issue_in(b, blk, sem):
        return pltpu.async_copy(tt_hbm.at[:, pl.ds(off_of(b), TBLK)], blk, sem)

    def wait_in(blk, sem):
        pltpu.make_async_copy(tt_hbm.at[:, pl.ds(0, TBLK)], blk, sem).wait()

    scat = lax.iota(jnp.int32, LANES) * DIM

    def transpose(blk, tr):
        for d in range(DIM):
            segs = [blk[d, pl.ds(vg * LANES, LANES)] for vg in range(TBLK // LANES)]
            for vg in range(TBLK // LANES):
                plsc.store_scatter(tr, [scat + (vg * LANES * DIM + d)], segs[vg])

    def issue_out(b, tr, sem):
        return pltpu.async_copy(tr, tl_hbm.at[pl.ds(off_of(b) * DIM, TBLK * DIM)], sem)

    def wait_out(tr, sem):
        pltpu.make_async_copy(tr, tl_hbm.at[pl.ds(0, TBLK * DIM)], sem).wait()

    issue_in(b0, blkA, s_iA)
    issue_out(b0, trA, s_oA)
    issue_out(b0 + 1, trB, s_oB)

    def pair(i, c):
        be = b0 + 2 * i
        issue_in(be + 1, blkB, s_iB)
        wait_out(trA, s_oA)
        wait_in(blkA, s_iA)
        transpose(blkA, trA)
        issue_out(be, trA, s_oA)
        issue_in(be + 2, blkA, s_iA)
        wait_out(trB, s_oB)
        wait_in(blkB, s_iB)
        transpose(blkB, trB)
        issue_out(be + 1, trB, s_oB)
        return c

    @pl.when(wid == 0)
    def _():
        pltpu.sync_copy(tail_hbm, tailv)
        pltpu.sync_copy(tailv, tl_hbm.at[pl.ds(VMAIN * DIM, NTAIL * DIM)])

    lax.fori_loop(0, BPW_A // 2, pair, 0)
    wait_in(blkA, s_iA)
    wait_out(trA, s_oA)
    wait_out(trB, s_oB)


_tr_kernel = pl.kernel(
    _tr_body,
    out_type=jax.ShapeDtypeStruct((VOCAB * DIM,), jnp.float32),
    mesh=plsc.VectorSubcoreMesh(core_axis_name="c", subcore_axis_name="s",
                                num_cores=NC, num_subcores=NS),
    scratch_types=[
        pltpu.VMEM((DIM, TBLK), jnp.float32),
        pltpu.VMEM((DIM, TBLK), jnp.float32),
        pltpu.VMEM((TBLK * DIM,), jnp.float32),
        pltpu.VMEM((TBLK * DIM,), jnp.float32),
        pltpu.VMEM((NTAIL * DIM,), jnp.float32),
        pltpu.SemaphoreType.DMA,
        pltpu.SemaphoreType.DMA,
        pltpu.SemaphoreType.DMA,
        pltpu.SemaphoreType.DMA,
    ],
    compiler_params=pltpu.CompilerParams(needs_layout_passes=False,
                                         use_tc_tiling_on_sc=True),
)

CH = 1280
PER_W = N // NW
NCH = PER_W // CH


def _sc_body(lidx_hbm, ridx_hbm, table_hbm,
             uu_hbm, uv_hbm, vv_hbm, al_hbm, be_hbm, ga_hbm,
             liA, riA, liB, riB, uA, vA, uB, vB,
             ouuA, ouvA, ovvA, oalA, obeA, ogaA,
             ouuB, ouvB, ovvB, oalB, obeB, ogaB,
             s_uA, s_vA, s_uB, s_vB, s_oA, s_oB):
    cid = lax.axis_index("c")
    sid = lax.axis_index("s")
    wid = sid * NC + cid
    base_w = wid * PER_W

    def base_of(c):
        return base_w + jnp.minimum(c, NCH - 1) * CH

    H = CH // 2

    def fetch(c, li, ri, u_v, v_v, su, sv):
        base = base_of(c)
        pltpu.sync_copy(lidx_hbm.at[pl.ds(base, CH)], li)
        pltpu.sync_copy(ridx_hbm.at[pl.ds(base, CH)], ri)
        pltpu.async_copy(table_hbm.at[li.at[pl.ds(0, H)]], u_v.at[pl.ds(0, H), :], su)
        pltpu.async_copy(table_hbm.at[li.at[pl.ds(H, H)]], u_v.at[pl.ds(H, H), :], su)
        pltpu.async_copy(table_hbm.at[ri.at[pl.ds(0, H)]], v_v.at[pl.ds(0, H), :], sv)
        pltpu.async_copy(table_hbm.at[ri.at[pl.ds(H, H)]], v_v.at[pl.ds(H, H), :], sv)

    def wait_fetch(u_v, v_v, su, sv):
        pltpu.make_async_copy(table_hbm.at[pl.ds(0, H), :], u_v.at[pl.ds(0, H), :], su).wait()
        pltpu.make_async_copy(table_hbm.at[pl.ds(0, H), :], u_v.at[pl.ds(H, H), :], su).wait()
        pltpu.make_async_copy(table_hbm.at[pl.ds(0, H), :], v_v.at[pl.ds(0, H), :], sv).wait()
        pltpu.make_async_copy(table_hbm.at[pl.ds(0, H), :], v_v.at[pl.ds(H, H), :], sv).wait()

    def compute(u_v, v_v, ouu, ouv, ovv, oal, obe, oga):
        def grp(g, c2):
            rows = pl.multiple_of(g * LANES, LANES) + lax.iota(jnp.int32, LANES)
            uu = jnp.zeros((LANES,), jnp.float32)
            vv = jnp.zeros((LANES,), jnp.float32)
            uv = jnp.zeros((LANES,), jnp.float32)
            for d in range(DIM):
                dcol = jnp.full((LANES,), d, jnp.int32)
                cu = plsc.load_gather(u_v, [rows, dcol])
                cv = plsc.load_gather(v_v, [rows, dcol])
                uu = uu + cu * cu
                vv = vv + cv * cv
                uv = uv + cu * cv
            alpha = 1.0 - uu
            alpha = jnp.where(alpha <= 0.0, EPS, alpha)
            beta = 1.0 - vv
            beta = jnp.where(beta <= 0.0, EPS, beta)
            gamma = 1.0 + 2.0 * (uu - 2.0 * uv + vv) / alpha / beta
            gamma = jnp.maximum(gamma, 1.0)
            s = pl.ds(pl.multiple_of(g * LANES, LANES), LANES)
            ouu[s] = uu
            ouv[s] = uv
            ovv[s] = vv
            oal[s] = alpha
            obe[s] = beta
            oga[s] = gamma
            return c2

        lax.fori_loop(0, CH // LANES, grp, 0)

    def issue_out(c, bufs, sem):
        dst = pl.ds(base_of(c), CH)
        cps = []
        for buf, hbm in zip(bufs, (uu_hbm, uv_hbm, vv_hbm, al_hbm, be_hbm, ga_hbm)):
            cps.append(pltpu.async_copy(buf, hbm.at[dst], sem))
        return cps

    bufsA = (ouuA, ouvA, ovvA, oalA, obeA, ogaA)
    bufsB = (ouuB, ouvB, ovvB, oalB, obeB, ogaB)

    fetch(0, liA, riA, uA, vA, s_uA, s_vA)

    def body(i, c):
        c0 = 2 * i
        fetch(c0 + 1, liB, riB, uB, vB, s_uB, s_vB)
        wait_fetch(uA, vA, s_uA, s_vA)
        compute(uA, vA, *bufsA)
        cpsA = issue_out(c0, bufsA, s_oA)
        fetch(c0 + 2, liA, riA, uA, vA, s_uA, s_vA)
        wait_fetch(uB, vB, s_uB, s_vB)
        compute(uB, vB, *bufsB)
        cpsB = issue_out(c0 + 1, bufsB, s_oB)
        for cp in cpsA:
            cp.wait()
        for cp in cpsB:
            cp.wait()
        return c

    lax.fori_loop(0, NCH // 2, body, 0)
    wait_fetch(uA, vA, s_uA, s_vA)


_sc_kernel = pl.kernel(
    _sc_body,
    out_type=tuple(jax.ShapeDtypeStruct((N,), jnp.float32) for _ in range(6)),
    mesh=plsc.VectorSubcoreMesh(core_axis_name="c", subcore_axis_name="s",
                                num_cores=NC, num_subcores=NS),
    scratch_types=(
        [pltpu.VMEM((CH,), jnp.int32)] * 4
        + [pltpu.VMEM((CH, DIM), jnp.float32)] * 4
        + [pltpu.VMEM((CH,), jnp.float32)] * 12
        + [pltpu.SemaphoreType.DMA] * 6
    ),
    compiler_params=pltpu.CompilerParams(needs_layout_passes=False,
                                         use_tc_tiling_on_sc=False),
)


def _arcosh_body(g_ref, d_ref):
    g = g_ref[...]
    d_ref[...] = jnp.log(g + jnp.sqrt(g * g - 1.0))


def kernel(left_idx, right_idx, table):
    lidx = left_idx.reshape(N).astype(jnp.int32)
    ridx = right_idx.reshape(N).astype(jnp.int32)
    tail = lax.slice(table, (VMAIN, 0), (VOCAB, DIM)).reshape(NTAIL * DIM)
    tl = _tr_kernel(table.T, tail)
    uu, uv, vv, alpha, beta, gamma = _sc_kernel(lidx, ridx, tl.reshape(VOCAB, DIM))
    g2d = gamma.reshape(N // 128, 128)
    dists = pl.pallas_call(
        _arcosh_body,
        out_shape=jax.ShapeDtypeStruct((N // 128, 128), jnp.float32),
    )(g2d)
    shp = (B, L)
    return (uu.reshape(shp), uv.reshape(shp), vv.reshape(shp),
            alpha.reshape(shp), beta.reshape(shp), gamma.reshape(shp),
            dists.reshape(shp))


# speedup vs baseline: 1.0408x; 1.0058x over previous
import functools

import jax
import jax.numpy as jnp
from jax import lax
from jax.experimental import pallas as pl
from jax.experimental.pallas import tpu as pltpu
from jax.experimental.pallas import tpu_sc as plsc

VOCAB = 1000000
DIM = 16
EPS = 1e-05
B = 16384
L = 50
N = B * L

NC, NS, LANES = 2, 16, 16
NW = NC * NS

TBLK = 256
VMAIN = (VOCAB // TBLK) * TBLK
NTAIL = VOCAB - VMAIN
MAXOFF_A = VMAIN - TBLK
BPW_A = 124


def _tr_body(tt_hbm, tail_hbm, tl_hbm, blkA, blkB, blkC, blkD,
             trA, trB, trC, trD, tailv,
             s_iA, s_iB, s_iC, s_iD, s_oA, s_oB, s_oC, s_oD):
    cid = lax.axis_index("c")
    sid = lax.axis_index("s")
    wid = sid * NC + cid
    b0 = wid * BPW_A

    def off_of(b):
        return pl.multiple_of(jnp.minimum(b * TBLK, MAXOFF_A), TBLK)

    def issue_in(b, blk, sem):
        return pltpu.async_copy(tt_hbm.at[:, pl.ds(off_of(b), TBLK)], blk, sem)

    def wait_in(blk, sem):
        pltpu.make_async_copy(tt_hbm.at[:, pl.ds(0, TBLK)], blk, sem).wait()

    scat = lax.iota(jnp.int32, LANES) * DIM

    NV = TBLK // LANES

    def transpose(blk, tr):
        prev = None
        for d in range(DIM):
            cur = []
            for vg in range(NV):
                cur.append(blk[d, pl.ds(vg * LANES, LANES)])
                if prev is not None:
                    plsc.store_scatter(tr, [scat + (vg * LANES * DIM + d - 1)],
                                       prev[vg])
            prev = cur
        for vg in range(NV):
            plsc.store_scatter(tr, [scat + (vg * LANES * DIM + DIM - 1)],
                               prev[vg])

    def issue_out(b, tr, sem):
        return pltpu.async_copy(tr, tl_hbm.at[pl.ds(off_of(b) * DIM, TBLK * DIM)], sem)

    def wait_out(tr, sem):
        pltpu.make_async_copy(tr, tl_hbm.at[pl.ds(0, TBLK * DIM)], sem).wait()

    blks = (blkA, blkB, blkC, blkD)
    trs = (trA, trB, trC, trD)
    sins = (s_iA, s_iB, s_iC, s_iD)
    souts = (s_oA, s_oB, s_oC, s_oD)

    for k in range(4):
        issue_in(b0 + k, blks[k], sins[k])
        issue_out(b0 + k, trs[k], souts[k])

    def quad(i, c):
        base = b0 + 4 * i
        for k in range(4):
            wait_out(trs[k], souts[k])
            wait_in(blks[k], sins[k])
            transpose(blks[k], trs[k])
            issue_out(base + k, trs[k], souts[k])
            issue_in(base + 4 + k, blks[k], sins[k])
        return c

    @pl.when(wid == 0)
    def _():
        pltpu.sync_copy(tail_hbm, tailv)
        pltpu.sync_copy(tailv, tl_hbm.at[pl.ds(VMAIN * DIM, NTAIL * DIM)])

    lax.fori_loop(0, BPW_A // 4, quad, 0)
    for k in range(4):
        wait_in(blks[k], sins[k])
        wait_out(trs[k], souts[k])


_tr_kernel = pl.kernel(
    _tr_body,
    out_type=jax.ShapeDtypeStruct((VOCAB * DIM,), jnp.float32),
    mesh=plsc.VectorSubcoreMesh(core_axis_name="c", subcore_axis_name="s",
                                num_cores=NC, num_subcores=NS),
    scratch_types=(
        [pltpu.VMEM((DIM, TBLK), jnp.float32)] * 4
        + [pltpu.VMEM((TBLK * DIM,), jnp.float32)] * 4
        + [pltpu.VMEM((NTAIL * DIM,), jnp.float32)]
        + [pltpu.SemaphoreType.DMA] * 8
    ),
    compiler_params=pltpu.CompilerParams(needs_layout_passes=False,
                                         use_tc_tiling_on_sc=True),
)

CH = 1280
PER_W = N // NW
NCH = PER_W // CH


def _sc_body(lidx_hbm, ridx_hbm, table_hbm,
             uu_hbm, uv_hbm, vv_hbm, al_hbm, be_hbm, ga_hbm,
             liA, riA, liB, riB, uA, vA, uB, vB,
             ouuA, ouvA, ovvA, oalA, obeA, ogaA,
             ouuB, ouvB, ovvB, oalB, obeB, ogaB,
             s_uA, s_vA, s_uB, s_vB, s_oA, s_oB):
    cid = lax.axis_index("c")
    sid = lax.axis_index("s")
    wid = sid * NC + cid
    base_w = wid * PER_W

    def base_of(c):
        return base_w + jnp.minimum(c, NCH - 1) * CH

    H = CH // 2

    def fetch(c, li, ri, u_v, v_v, su, sv):
        base = base_of(c)
        pltpu.sync_copy(lidx_hbm.at[pl.ds(base, CH)], li)
        pltpu.sync_copy(ridx_hbm.at[pl.ds(base, CH)], ri)
        pltpu.async_copy(table_hbm.at[li.at[pl.ds(0, H)]], u_v.at[pl.ds(0, H), :], su)
        pltpu.async_copy(table_hbm.at[li.at[pl.ds(H, H)]], u_v.at[pl.ds(H, H), :], su)
        pltpu.async_copy(table_hbm.at[ri.at[pl.ds(0, H)]], v_v.at[pl.ds(0, H), :], sv)
        pltpu.async_copy(table_hbm.at[ri.at[pl.ds(H, H)]], v_v.at[pl.ds(H, H), :], sv)

    def wait_fetch(u_v, v_v, su, sv):
        pltpu.make_async_copy(table_hbm.at[pl.ds(0, H), :], u_v.at[pl.ds(0, H), :], su).wait()
        pltpu.make_async_copy(table_hbm.at[pl.ds(0, H), :], u_v.at[pl.ds(H, H), :], su).wait()
        pltpu.make_async_copy(table_hbm.at[pl.ds(0, H), :], v_v.at[pl.ds(0, H), :], sv).wait()
        pltpu.make_async_copy(table_hbm.at[pl.ds(0, H), :], v_v.at[pl.ds(H, H), :], sv).wait()

    def compute(u_v, v_v, ouu, ouv, ovv, oal, obe, oga):
        def grp(g, c2):
            rows = pl.multiple_of(g * LANES, LANES) + lax.iota(jnp.int32, LANES)
            uu = jnp.zeros((LANES,), jnp.float32)
            vv = jnp.zeros((LANES,), jnp.float32)
            uv = jnp.zeros((LANES,), jnp.float32)
            for d in range(DIM):
                dcol = jnp.full((LANES,), d, jnp.int32)
                cu = plsc.load_gather(u_v, [rows, dcol])
                cv = plsc.load_gather(v_v, [rows, dcol])
                uu = uu + cu * cu
                vv = vv + cv * cv
                uv = uv + cu * cv
            alpha = 1.0 - uu
            alpha = jnp.where(alpha <= 0.0, EPS, alpha)
            beta = 1.0 - vv
            beta = jnp.where(beta <= 0.0, EPS, beta)
            gamma = 1.0 + 2.0 * (uu - 2.0 * uv + vv) / alpha / beta
            gamma = jnp.maximum(gamma, 1.0)
            s = pl.ds(pl.multiple_of(g * LANES, LANES), LANES)
            ouu[s] = uu
            ouv[s] = uv
            ovv[s] = vv
            oal[s] = alpha
            obe[s] = beta
            oga[s] = gamma
            return c2

        lax.fori_loop(0, CH // LANES, grp, 0)

    def issue_out(c, bufs, sem):
        dst = pl.ds(base_of(c), CH)
        cps = []
        for buf, hbm in zip(bufs, (uu_hbm, uv_hbm, vv_hbm, al_hbm, be_hbm, ga_hbm)):
            cps.append(pltpu.async_copy(buf, hbm.at[dst], sem))
        return cps

    bufsA = (ouuA, ouvA, ovvA, oalA, obeA, ogaA)
    bufsB = (ouuB, ouvB, ovvB, oalB, obeB, ogaB)

    fetch(0, liA, riA, uA, vA, s_uA, s_vA)

    def body(i, c):
        c0 = 2 * i
        fetch(c0 + 1, liB, riB, uB, vB, s_uB, s_vB)
        wait_fetch(uA, vA, s_uA, s_vA)
        compute(uA, vA, *bufsA)
        cpsA = issue_out(c0, bufsA, s_oA)
        fetch(c0 + 2, liA, riA, uA, vA, s_uA, s_vA)
        wait_fetch(uB, vB, s_uB, s_vB)
        compute(uB, vB, *bufsB)
        cpsB = issue_out(c0 + 1, bufsB, s_oB)
        for cp in cpsA:
            cp.wait()
        for cp in cpsB:
            cp.wait()
        return c

    lax.fori_loop(0, NCH // 2, body, 0)
    wait_fetch(uA, vA, s_uA, s_vA)


_sc_kernel = pl.kernel(
    _sc_body,
    out_type=tuple(jax.ShapeDtypeStruct((N,), jnp.float32) for _ in range(6)),
    mesh=plsc.VectorSubcoreMesh(core_axis_name="c", subcore_axis_name="s",
                                num_cores=NC, num_subcores=NS),
    scratch_types=(
        [pltpu.VMEM((CH,), jnp.int32)] * 4
        + [pltpu.VMEM((CH, DIM), jnp.float32)] * 4
        + [pltpu.VMEM((CH,), jnp.float32)] * 12
        + [pltpu.SemaphoreType.DMA] * 6
    ),
    compiler_params=pltpu.CompilerParams(needs_layout_passes=False,
                                         use_tc_tiling_on_sc=False),
)


def _arcosh_body(g_ref, d_ref):
    g = g_ref[...]
    d_ref[...] = jnp.log(g + jnp.sqrt(g * g - 1.0))


def kernel(left_idx, right_idx, table):
    lidx = left_idx.reshape(N).astype(jnp.int32)
    ridx = right_idx.reshape(N).astype(jnp.int32)
    tail = lax.slice(table, (VMAIN, 0), (VOCAB, DIM)).reshape(NTAIL * DIM)
    tl = _tr_kernel(table.T, tail)
    uu, uv, vv, alpha, beta, gamma = _sc_kernel(lidx, ridx, tl.reshape(VOCAB, DIM))
    g2d = gamma.reshape(N // 128, 128)
    dists = pl.pallas_call(
        _arcosh_body,
        out_shape=jax.ShapeDtypeStruct((N // 128, 128), jnp.float32),
    )(g2d)
    shp = (B, L)
    return (uu.reshape(shp), uv.reshape(shp), vv.reshape(shp),
            alpha.reshape(shp), beta.reshape(shp), gamma.reshape(shp),
            dists.reshape(shp))


# speedup vs baseline: 1.0419x; 1.0010x over previous
import jax
import jax.numpy as jnp
from jax import lax
from jax.experimental import pallas as pl
from jax.experimental.pallas import tpu as pltpu
from jax.experimental.pallas import tpu_sc as plsc

VOCAB = 1000000
DIM = 16
EPS = 1e-05
B = 16384
L = 50
N = B * L

NC, NS, LANES = 2, 16, 16
NW = NC * NS

TBLK = 256
VMAIN = (VOCAB // TBLK) * TBLK
NTAIL = VOCAB - VMAIN
MAXOFF_A = VMAIN - TBLK
BPW_A = 124


def _tr_body(tt_hbm, tail_hbm, tl_hbm, blkA, blkB, blkC, blkD,
             trA, trB, trC, trD, tailv,
             s_iA, s_iB, s_iC, s_iD, s_oA, s_oB, s_oC, s_oD):
    cid = lax.axis_index("c")
    sid = lax.axis_index("s")
    wid = sid * NC + cid
    b0 = wid * BPW_A

    def off_of(b):
        return pl.multiple_of(jnp.minimum(b * TBLK, MAXOFF_A), TBLK)

    def issue_in(b, blk, sem):
        return pltpu.async_copy(tt_hbm.at[:, pl.ds(off_of(b), TBLK)], blk, sem)

    def wait_in(blk, sem):
        pltpu.make_async_copy(tt_hbm.at[:, pl.ds(0, TBLK)], blk, sem).wait()

    scat = lax.iota(jnp.int32, LANES) * DIM

    NV = TBLK // LANES

    def transpose(blk, tr):
        prev = None
        for d in range(DIM):
            cur = []
            for vg in range(NV):
                cur.append(blk[d, pl.ds(vg * LANES, LANES)])
                if prev is not None:
                    plsc.store_scatter(tr, [scat + (vg * LANES * DIM + d - 1)],
                                       prev[vg])
            prev = cur
        for vg in range(NV):
            plsc.store_scatter(tr, [scat + (vg * LANES * DIM + DIM - 1)],
                               prev[vg])

    def issue_out(b, tr, sem):
        return pltpu.async_copy(tr, tl_hbm.at[pl.ds(off_of(b) * DIM, TBLK * DIM)], sem)

    def wait_out(tr, sem):
        pltpu.make_async_copy(tr, tl_hbm.at[pl.ds(0, TBLK * DIM)], sem).wait()

    blks = (blkA, blkB, blkC, blkD)
    trs = (trA, trB, trC, trD)
    sins = (s_iA, s_iB, s_iC, s_iD)
    souts = (s_oA, s_oB, s_oC, s_oD)

    for k in range(4):
        issue_in(b0 + k, blks[k], sins[k])
        issue_out(b0 + k, trs[k], souts[k])

    def quad(i, c):
        base = b0 + 4 * i
        for k in range(4):
            wait_out(trs[k], souts[k])
            wait_in(blks[k], sins[k])
            transpose(blks[k], trs[k])
            issue_out(base + k, trs[k], souts[k])
            issue_in(base + 4 + k, blks[k], sins[k])
        return c

    @pl.when(wid == 0)
    def _():
        pltpu.sync_copy(tail_hbm, tailv)
        pltpu.sync_copy(tailv, tl_hbm.at[pl.ds(VMAIN * DIM, NTAIL * DIM)])

    lax.fori_loop(0, BPW_A // 4, quad, 0)
    for k in range(4):
        wait_in(blks[k], sins[k])
        wait_out(trs[k], souts[k])


_tr_kernel = pl.kernel(
    _tr_body,
    out_type=jax.ShapeDtypeStruct((VOCAB * DIM,), jnp.float32),
    mesh=plsc.VectorSubcoreMesh(core_axis_name="c", subcore_axis_name="s",
                                num_cores=NC, num_subcores=NS),
    scratch_types=(
        [pltpu.VMEM((DIM, TBLK), jnp.float32)] * 4
        + [pltpu.VMEM((TBLK * DIM,), jnp.float32)] * 4
        + [pltpu.VMEM((NTAIL * DIM,), jnp.float32)]
        + [pltpu.SemaphoreType.DMA] * 8
    ),
    compiler_params=pltpu.CompilerParams(needs_layout_passes=False,
                                         use_tc_tiling_on_sc=True),
)

CH = 1280
PER_W = N // NW
NCH = PER_W // CH


def _sc_body(lidx_hbm, ridx_hbm, table_hbm,
             uu_hbm, uv_hbm, vv_hbm, al_hbm, be_hbm, ga_hbm,
             liA, riA, liB, riB, uA, vA, uB, vB,
             ouuA, ouvA, ovvA, oalA, obeA, ogaA,
             ouuB, ouvB, ovvB, oalB, obeB, ogaB,
             s_uA, s_vA, s_uB, s_vB, s_oA, s_oB):
    cid = lax.axis_index("c")
    sid = lax.axis_index("s")
    wid = sid * NC + cid
    base_w = wid * PER_W

    def base_of(c):
        return base_w + jnp.minimum(c, NCH - 1) * CH

    H = CH // 2

    def fetch(c, li, ri, u_v, v_v, su, sv):
        base = base_of(c)
        pltpu.sync_copy(lidx_hbm.at[pl.ds(base, CH)], li)
        pltpu.sync_copy(ridx_hbm.at[pl.ds(base, CH)], ri)
        pltpu.async_copy(table_hbm.at[li.at[pl.ds(0, H)]], u_v.at[pl.ds(0, H), :], su)
        pltpu.async_copy(table_hbm.at[li.at[pl.ds(H, H)]], u_v.at[pl.ds(H, H), :], su)
        pltpu.async_copy(table_hbm.at[ri.at[pl.ds(0, H)]], v_v.at[pl.ds(0, H), :], sv)
        pltpu.async_copy(table_hbm.at[ri.at[pl.ds(H, H)]], v_v.at[pl.ds(H, H), :], sv)

    def wait_fetch(u_v, v_v, su, sv):
        pltpu.make_async_copy(table_hbm.at[pl.ds(0, H), :], u_v.at[pl.ds(0, H), :], su).wait()
        pltpu.make_async_copy(table_hbm.at[pl.ds(0, H), :], u_v.at[pl.ds(H, H), :], su).wait()
        pltpu.make_async_copy(table_hbm.at[pl.ds(0, H), :], v_v.at[pl.ds(0, H), :], sv).wait()
        pltpu.make_async_copy(table_hbm.at[pl.ds(0, H), :], v_v.at[pl.ds(H, H), :], sv).wait()

    def compute(u_v, v_v, ouu, ouv, ovv, oal, obe, oga):
        def grp(g, c2):
            rows = pl.multiple_of(g * LANES, LANES) + lax.iota(jnp.int32, LANES)
            uu = jnp.zeros((LANES,), jnp.float32)
            vv = jnp.zeros((LANES,), jnp.float32)
            uv = jnp.zeros((LANES,), jnp.float32)
            for d in range(DIM):
                dcol = jnp.full((LANES,), d, jnp.int32)
                cu = plsc.load_gather(u_v, [rows, dcol])
                cv = plsc.load_gather(v_v, [rows, dcol])
                uu = uu + cu * cu
                vv = vv + cv * cv
                uv = uv + cu * cv
            alpha = 1.0 - uu
            alpha = jnp.where(alpha <= 0.0, EPS, alpha)
            beta = 1.0 - vv
            beta = jnp.where(beta <= 0.0, EPS, beta)
            gamma = 1.0 + 2.0 * (uu - 2.0 * uv + vv) / alpha / beta
            gamma = jnp.maximum(gamma, 1.0)
            s = pl.ds(pl.multiple_of(g * LANES, LANES), LANES)
            ouu[s] = uu
            ouv[s] = uv
            ovv[s] = vv
            oal[s] = alpha
            obe[s] = beta
            oga[s] = gamma
            return c2

        lax.fori_loop(0, CH // LANES, grp, 0)

    def issue_out(c, bufs, sem):
        dst = pl.ds(base_of(c), CH)
        cps = []
        for buf, hbm in zip(bufs, (uu_hbm, uv_hbm, vv_hbm, al_hbm, be_hbm, ga_hbm)):
            cps.append(pltpu.async_copy(buf, hbm.at[dst], sem))
        return cps

    bufsA = (ouuA, ouvA, ovvA, oalA, obeA, ogaA)
    bufsB = (ouuB, ouvB, ovvB, oalB, obeB, ogaB)

    fetch(0, liA, riA, uA, vA, s_uA, s_vA)

    def body(i, c):
        c0 = 2 * i
        fetch(c0 + 1, liB, riB, uB, vB, s_uB, s_vB)
        wait_fetch(uA, vA, s_uA, s_vA)
        compute(uA, vA, *bufsA)
        cpsA = issue_out(c0, bufsA, s_oA)
        fetch(c0 + 2, liA, riA, uA, vA, s_uA, s_vA)
        wait_fetch(uB, vB, s_uB, s_vB)
        compute(uB, vB, *bufsB)
        cpsB = issue_out(c0 + 1, bufsB, s_oB)
        for cp in cpsA:
            cp.wait()
        for cp in cpsB:
            cp.wait()
        return c

    lax.fori_loop(0, NCH // 2, body, 0)
    wait_fetch(uA, vA, s_uA, s_vA)


_sc_kernel = pl.kernel(
    _sc_body,
    out_type=tuple(jax.ShapeDtypeStruct((N,), jnp.float32) for _ in range(6)),
    mesh=plsc.VectorSubcoreMesh(core_axis_name="c", subcore_axis_name="s",
                                num_cores=NC, num_subcores=NS),
    scratch_types=(
        [pltpu.VMEM((CH,), jnp.int32)] * 4
        + [pltpu.VMEM((CH, DIM), jnp.float32)] * 4
        + [pltpu.VMEM((CH,), jnp.float32)] * 12
        + [pltpu.SemaphoreType.DMA] * 6
    ),
    compiler_params=pltpu.CompilerParams(needs_layout_passes=False,
                                         use_tc_tiling_on_sc=False),
)


def _arcosh_body(g_ref, d_ref):
    g = g_ref[...]
    d_ref[...] = jnp.log(g + jnp.sqrt(g * g - 1.0))


def kernel(left_idx, right_idx, table):
    lidx = left_idx.reshape(N).astype(jnp.int32)
    ridx = right_idx.reshape(N).astype(jnp.int32)
    tail = lax.slice(table, (VMAIN, 0), (VOCAB, DIM)).reshape(NTAIL * DIM)
    tl = _tr_kernel(table.T, tail)
    uu, uv, vv, alpha, beta, gamma = _sc_kernel(lidx, ridx, tl.reshape(VOCAB, DIM))
    g2d = gamma.reshape(N // 128, 128)
    dists = pl.pallas_call(
        _arcosh_body,
        out_shape=jax.ShapeDtypeStruct((N // 128, 128), jnp.float32),
    )(g2d)
    shp = (B, L)
    return (uu.reshape(shp), uv.reshape(shp), vv.reshape(shp),
            alpha.reshape(shp), beta.reshape(shp), gamma.reshape(shp),
            dists.reshape(shp))
